# Initial kernel scaffold; baseline (speedup 1.0000x reference)
import jax
import jax.numpy as jnp
from jax.experimental import pallas as pl


def kernel(X, edge_index, W, attention_w):
    raise NotImplementedError("write your pallas kernel here")


# speedup vs baseline: 2.8839x; 2.8839x over previous
import functools

import jax
import jax.numpy as jnp
from jax import lax
from jax.experimental import pallas as pl
from jax.experimental.pallas import tpu as pltpu
from jax.experimental.pallas import tpu_sc as plsc

_NC = 2
_NS = 16
_NW = _NC * _NS
_L = 16
_C = 80

_PERM_DNUMS = lax.GatherDimensionNumbers(
    offset_dims=(), collapsed_slice_dims=(0,), start_index_map=(0,))


def _lane_sum_splat(x):
    iota = lax.iota(jnp.int32, _L)
    for sh in (8, 4, 2, 1):
        idx = jnp.bitwise_xor(iota, sh)
        x = x + lax.gather(x, idx[:, None], _PERM_DNUMS, (1,),
                           mode=lax.GatherScatterMode.PROMISE_IN_BOUNDS)
    return x


def _mm_body(x_ref, w_ref, o_ref):
    o_ref[...] = jnp.dot(x_ref[...], w_ref[...],
                         preferred_element_type=jnp.float32)


def _matmul(X, W):
    N, D = X.shape
    BN = 1000
    return pl.pallas_call(
        _mm_body,
        grid=(N // BN,),
        in_specs=[
            pl.BlockSpec((BN, D), lambda i: (i, 0)),
            pl.BlockSpec((D, D), lambda i: (0, 0)),
        ],
        out_specs=pl.BlockSpec((BN, D), lambda i: (i, 0)),
        out_shape=jax.ShapeDtypeStruct((N, D), jnp.float32),
    )(X, W)


def _combine_body(hp_ref, rs_ref, o_ref):
    h = hp_ref[0] + hp_ref[1]
    r = rs_ref[0, :, 0:1] + rs_ref[1, :, 0:1]
    o_ref[...] = h / r


def _combine(h_part, rs_part):
    _, N, D = h_part.shape
    BN = next(b for b in (1024, 512, 256, 128) if N % b == 0)
    return pl.pallas_call(
        _combine_body,
        grid=(N // BN,),
        in_specs=[
            pl.BlockSpec((2, BN, D), lambda i: (0, i, 0)),
            pl.BlockSpec((2, BN, _L), lambda i: (0, i, 0)),
        ],
        out_specs=pl.BlockSpec((BN, D), lambda i: (i, 0)),
        out_shape=jax.ShapeDtypeStruct((N, D), jnp.float32),
    )(h_part, rs_part)


def _make_sc_agg(Np, E, D):
    per_w = E // _NW
    nchunks = per_w // _C
    rows_per_tile = Np // _NS
    mesh = plsc.VectorSubcoreMesh(core_axis_name="c", subcore_axis_name="s")

    @functools.partial(
        pl.kernel,
        mesh=mesh,
        out_type=[
            jax.ShapeDtypeStruct((_NC * Np, D), jnp.float32),
            jax.ShapeDtypeStruct((_NC * Np // 8, D), jnp.float32),
        ],
        scratch_types=[
            pltpu.VMEM((_C,), jnp.int32),
            pltpu.VMEM((_C,), jnp.int32),
            pltpu.VMEM((_C,), jnp.int32),
            pltpu.VMEM((_C, D), jnp.float32),
            pltpu.VMEM((_C, D), jnp.float32),
            pltpu.VMEM((_C, D), jnp.float32),
            pltpu.VMEM((_L,), jnp.float32),
            pltpu.VMEM_SHARED((Np, D), jnp.float32),
            pltpu.VMEM_SHARED((Np // 8, D), jnp.float32),
            pltpu.SemaphoreType.DMA,
            pltpu.SemaphoreType.DMA,
        ],
    )
    def sc_agg(xp_hbm, dst_hbm, src_hbm, aw_hbm,
               h_out, rs_out,
               dst_v, src_v, idx8_v, drows, srows, abuf, awv,
               h_sh, rs_sh,
               sem1, sem2):
        cid = lax.axis_index("c")
        sid = lax.axis_index("s")
        wid = cid * _NS + sid

        r0 = sid * rows_per_tile
        nslabs = rows_per_tile // _C
        pltpu.sync_copy(aw_hbm, awv)

        iota = lax.iota(jnp.int32, _L)
        rs_rows = Np // 8 // _NS
        rrs0 = sid * rs_rows

        def zrow_body(r, carry):
            z = jnp.zeros((_L,), jnp.float32)
            for k in range(D // _L):
                srows[r, pl.ds(k * _L, _L)] = z
            return carry

        lax.fori_loop(0, _C, zrow_body, 0)

        def zslab_body(b, carry):
            pltpu.sync_copy(srows, h_sh.at[pl.ds(r0 + b * _C, _C)])
            return carry

        lax.fori_loop(0, nslabs, zslab_body, 0)
        pltpu.sync_copy(srows.at[pl.ds(0, rs_rows)],
                        rs_sh.at[pl.ds(rrs0, rs_rows)])
        plsc.subcore_barrier()

        def chunk_body(j, carry):
            base = wid * per_w + j * _C
            pltpu.sync_copy(dst_hbm.at[pl.ds(base, _C)], dst_v)
            pltpu.sync_copy(src_hbm.at[pl.ds(base, _C)], src_v)
            pltpu.async_copy(xp_hbm.at[dst_v], drows, sem1).wait()
            pltpu.async_copy(xp_hbm.at[src_v], srows, sem2).wait()
            aw = awv[...]

            def edge_body(e, c2):
                acc = drows[e, pl.ds(0, _L)] * srows[e, pl.ds(0, _L)]
                for k in range(1, D // _L):
                    acc = acc + (drows[e, pl.ds(k * _L, _L)]
                                 * srows[e, pl.ds(k * _L, _L)])
                s = _lane_sum_splat(acc)
                att = jnp.exp(s * aw)
                for k in range(D // _L):
                    srows[e, pl.ds(k * _L, _L)] = (
                        srows[e, pl.ds(k * _L, _L)] * att)
                gb = (e // _L) * _L
                dst_grp = dst_v[pl.ds(gb, _L)]
                lane = jnp.full((_L,), e - gb, jnp.int32)
                dst_splat = lax.gather(
                    dst_grp, lane[:, None], _PERM_DNUMS, (1,),
                    mode=lax.GatherScatterMode.PROMISE_IN_BOUNDS)
                slot = jnp.bitwise_and(dst_splat, 7) * _L
                for k in range(D // _L):
                    m = (1 - jnp.minimum(jnp.abs(slot - k * _L), 1)
                         ).astype(jnp.float32)
                    abuf[e, pl.ds(k * _L, _L)] = att * m
                return c2

            lax.fori_loop(0, _C, edge_body, 0)

            def idx8_body(g, c2):
                d16 = dst_v[pl.ds(g * _L, _L)]
                idx8_v[pl.ds(g * _L, _L)] = lax.shift_right_logical(d16, 3)
                return c2

            lax.fori_loop(0, _C // _L, idx8_body, 0)
            pltpu.sync_copy(srows, h_sh.at[dst_v], add=True)
            pltpu.sync_copy(abuf, rs_sh.at[idx8_v], add=True)
            return carry

        lax.fori_loop(0, nchunks, chunk_body, 0)
        plsc.subcore_barrier()

        o0 = cid * Np + r0

        def drain_body(b, carry):
            pltpu.sync_copy(h_sh.at[pl.ds(r0 + b * _C, _C)], srows)
            pltpu.sync_copy(srows, h_out.at[pl.ds(o0 + b * _C, _C)])
            return carry

        lax.fori_loop(0, nslabs, drain_body, 0)
        pltpu.sync_copy(rs_sh.at[pl.ds(rrs0, rs_rows)],
                        srows.at[pl.ds(0, rs_rows)])
        pltpu.sync_copy(srows.at[pl.ds(0, rs_rows)],
                        rs_out.at[pl.ds(cid * (Np // 8) + rrs0, rs_rows)])

    return sc_agg


def kernel(X, edge_index, W, attention_w):
    N, D = X.shape
    E = edge_index.shape[1]
    Np = ((N + _NS * _C - 1) // (_NS * _C)) * (_NS * _C)
    dst = edge_index[0].astype(jnp.int32)
    src = edge_index[1].astype(jnp.int32)
    aw16 = jnp.broadcast_to(attention_w.astype(jnp.float32), (_L,))

    xp = _matmul(X, W)
    h_part, rs_part = _make_sc_agg(Np, E, D)(xp, dst, src, aw16)
    h_part = h_part.reshape(_NC, Np, D)
    rs_part = rs_part.reshape(_NC, Np, _L)
    return _combine(h_part, rs_part)[:N]
